# baseline (speedup 1.0000x reference)
import functools

import jax
import jax.numpy as jnp
from jax import lax
from jax.experimental import pallas as pl
from jax.experimental.pallas import tpu as pltpu
from jax.experimental.pallas import tpu_sc as plsc

H, W, KS = 2048, 2048, 8
N = 65536
E = H * W * KS
VE = N * KS
NC, NS, L = 2, 16, 16
CHUNK = 524288
NCHUNK = E // CHUNK
CPS = NCHUNK // NC
BSZ = CHUNK + NS * 1024
SSZ = CHUNK // NS
PTS = N // NS
ZROWS = 8192
G = 128
TRASH = CHUNK

_mesh = plsc.VectorSubcoreMesh(
    core_axis_name="c", subcore_axis_name="s", num_cores=NC, num_subcores=NS
)


def _scalar(v16, i):
    return lax.squeeze(lax.slice(v16, (i,), (i + 1,)), (0,))


@functools.partial(
    pl.kernel,
    out_type=jax.ShapeDtypeStruct((E,), jnp.float32),
    mesh=_mesh,
    scratch_types=[
        pltpu.VMEM((PTS,), jnp.int32),
        pltpu.VMEM((PTS,), jnp.int32),
        pltpu.VMEM((PTS,), jnp.int32),
        pltpu.VMEM((PTS,), jnp.int32),
        pltpu.VMEM((PTS,), jnp.int32),
        pltpu.VMEM((CPS * L,), jnp.int32),
        pltpu.VMEM((CPS + L,), jnp.int32),
        pltpu.VMEM((G,), jnp.int32),
        pltpu.VMEM((G,), jnp.int32),
        pltpu.VMEM((KS, G), jnp.float32),
        pltpu.VMEM((ZROWS,), jnp.float32),
        pltpu.VMEM_SHARED((2 * BSZ,), jnp.float32),
        pltpu.SemaphoreType.DMA,
        pltpu.SemaphoreType.DMA,
        pltpu.SemaphoreType.DMA,
        pltpu.SemaphoreType.DMA,
    ],
    compiler_params=pltpu.CompilerParams(needs_layout_passes=False),
)
def _scatter_kernel(xcol, ycol, vals1, out1, xs_v, ys_v, eb_v, vb_v, bli,
                    roff, binst, eidx, vidx, crows, zbuf, acc,
                    gsem, ssem, zsem, wsem):
    c = lax.axis_index("c")
    s = lax.axis_index("s")
    iota = lax.iota(jnp.int32, L)
    zero_i = jnp.zeros((L,), jnp.int32)
    zero_f = jnp.zeros((L,), jnp.float32)

    pltpu.sync_copy(xcol.at[pl.ds(s * PTS, PTS)], xs_v)
    pltpu.sync_copy(ycol.at[pl.ds(s * PTS, PTS)], ys_v)

    def zh(i, _):
        plsc.store_scatter(roff, [iota + i * L], zero_i)
        return 0

    lax.fori_loop(0, CPS, zh, 0)

    cb0 = c * CPS * CHUNK

    def mk_base(i, _):
        lanes = iota + i * L
        xs = plsc.load_gather(xs_v, [lanes])
        ys = plsc.load_gather(ys_v, [lanes])
        eb = xs * (W * KS) + ((ys >> 7) << 10) + (ys & 127)
        plsc.store_scatter(eb_v, [lanes], eb)
        p = lanes + s * PTS
        plsc.store_scatter(vb_v, [lanes], ((p >> 7) << 10) + (p & 127))
        cid = (eb - cb0) >> 19
        m = (cid >= 0) & (cid < CPS)
        hidx = (cid << 4) + iota
        h = plsc.load_gather(roff, [hidx], mask=m)
        plsc.store_scatter(roff, [hidx], h + 1, mask=m)
        return 0

    lax.fori_loop(0, PTS // L, mk_base, 0)

    base = jnp.int32(0)
    for b in range(CPS):
        plsc.store_scatter(binst, [zero_i + b], zero_i + base,
                           mask=(iota == 0))
        row = plsc.load_gather(roff, [zero_i + (b << 4) + iota])
        cs = plsc.cumsum(row)
        plsc.store_scatter(roff, [zero_i + (b << 4) + iota],
                           base + cs - row)
        base = base + _scalar(cs, L - 1)
    plsc.store_scatter(binst, [zero_i + CPS], zero_i + base,
                       mask=(iota == 0))

    def binify(i, _):
        lanes = iota + i * L
        eb = plsc.load_gather(eb_v, [lanes])
        cid = (eb - cb0) >> 19
        m = (cid >= 0) & (cid < CPS)
        hidx = (cid << 4) + iota
        pos = plsc.load_gather(roff, [hidx], mask=m)
        plsc.store_scatter(roff, [hidx], pos + 1, mask=m)
        plsc.store_scatter(bli, [pos], lanes, mask=m)
        return 0

    lax.fori_loop(0, PTS // L, binify, 0)

    def mk_zero(i, _):
        plsc.store_scatter(zbuf, [iota + i * L], zero_f)
        return 0

    lax.fori_loop(0, ZROWS // L, mk_zero, 0)

    def z0(j, _):
        pltpu.sync_copy(zbuf, acc.at[pl.ds(s * SSZ + j * ZROWS, ZROWS)])
        pltpu.sync_copy(zbuf, acc.at[pl.ds(BSZ + s * SSZ + j * ZROWS, ZROWS)])
        return 0

    lax.fori_loop(0, SSZ // ZROWS, z0, 0)

    pltpu.async_copy(acc.at[pl.ds(s * SSZ, SSZ)],
                     out1.at[pl.ds(cb0 + s * SSZ, SSZ)], wsem)
    pltpu.async_copy(acc.at[pl.ds(BSZ + s * SSZ, SSZ)],
                     out1.at[pl.ds(cb0 + CHUNK + s * SSZ, SSZ)], wsem)

    KPAD = (KS - 1) * G

    def expand(start, end, cb, g):
        for j in range(G // L):
            lanes = start + g * G + j * L + iota
            valid = lanes < end
            li = plsc.load_gather(bli, [lanes], mask=valid)
            e16 = plsc.load_gather(eb_v, [li], mask=valid) - cb
            v16 = plsc.load_gather(vb_v, [li], mask=valid)
            e16 = jnp.where(valid, e16, TRASH + s * 1024)
            v16 = jnp.where(valid, v16, s * (PTS * KS))
            plsc.store_scatter(eidx, [j * L + iota], e16)
            plsc.store_scatter(vidx, [j * L + iota], v16)

    def chunk_body(lc, _):
        cb = cb0 + lc * CHUNK
        bb = (lc & 1) * BSZ
        st16 = plsc.load_gather(binst, [zero_i + lc])
        en16 = plsc.load_gather(binst, [zero_i + lc + 1])
        start, end = _scalar(st16, 0), _scalar(en16, 0)
        lcp = jnp.maximum(lc - 2, 0)
        pst16 = plsc.load_gather(binst, [zero_i + lcp])
        pen16 = plsc.load_gather(binst, [zero_i + lcp + 1])
        pstart, pend = _scalar(pst16, 0), _scalar(pen16, 0)
        pcnt = jnp.where(lc > 1, pend - pstart, 0)

        pltpu.make_async_copy(acc.at[pl.ds(s * SSZ, SSZ)],
                              out1.at[pl.ds(s * SSZ, SSZ)], wsem).wait()
        plsc.subcore_barrier()

        def zt(g, _):
            expand(pstart, pend, cb - 2 * CHUNK, g)
            for k in range(KS):
                pltpu.async_copy(
                    zbuf.at[pl.ds(0, G)],
                    acc.at[pl.ds(bb + k * G, BSZ - KPAD)].at[eidx], zsem)
            for k in range(KS):
                pltpu.make_async_copy(
                    zbuf.at[pl.ds(0, G)],
                    acc.at[pl.ds(bb + k * G, BSZ - KPAD)].at[eidx],
                    zsem).wait()
            return 0

        lax.fori_loop(0, (pcnt + G - 1) >> 7, zt, 0)

        plsc.subcore_barrier()

        def accum(g, _):
            expand(start, end, cb, g)
            for k in range(KS):
                pltpu.async_copy(
                    vals1.at[pl.ds(k * G, VE - KPAD)].at[vidx],
                    crows.at[k], gsem)
            for k in range(KS):
                pltpu.make_async_copy(
                    vals1.at[pl.ds(k * G, VE - KPAD)].at[vidx],
                    crows.at[k], gsem).wait()
            for k in range(KS):
                pltpu.async_copy(
                    crows.at[k],
                    acc.at[pl.ds(bb + k * G, BSZ - KPAD)].at[eidx], ssem,
                    add=True)
            for k in range(KS):
                pltpu.make_async_copy(
                    crows.at[k],
                    acc.at[pl.ds(bb + k * G, BSZ - KPAD)].at[eidx],
                    ssem).wait()
            return 0

        lax.fori_loop(0, (end - start + G - 1) >> 7, accum, 0)

        plsc.subcore_barrier()

        pltpu.async_copy(acc.at[pl.ds(bb + s * SSZ, SSZ)],
                         out1.at[pl.ds(cb + s * SSZ, SSZ)], wsem)
        return 0

    lax.fori_loop(0, CPS, chunk_body, 0)

    pltpu.make_async_copy(acc.at[pl.ds(s * SSZ, SSZ)],
                          out1.at[pl.ds(s * SSZ, SSZ)], wsem).wait()
    pltpu.make_async_copy(acc.at[pl.ds(s * SSZ, SSZ)],
                          out1.at[pl.ds(s * SSZ, SSZ)], wsem).wait()


def kernel(coords, vals):
    xcol = coords[:, 0]
    ycol = coords[:, 1]
    vals1 = vals.reshape(N // 128, 128, KS).transpose(0, 2, 1).reshape(VE)
    out1 = _scatter_kernel(xcol, ycol, vals1)
    return out1.reshape(H, W // 128, KS, 128).transpose(0, 1, 3, 2).reshape(
        H, W, KS)


# speedup vs baseline: 1.0171x; 1.0171x over previous
import functools

import jax
import jax.numpy as jnp
from jax import lax
from jax.experimental import pallas as pl
from jax.experimental.pallas import tpu as pltpu
from jax.experimental.pallas import tpu_sc as plsc

H, W, KS = 2048, 2048, 8
N = 65536
E = H * W * KS
VE = N * KS
NC, NS, L = 2, 16, 16
CHUNK = 1048576
NCHUNK = E // CHUNK
CPS = NCHUNK // NC
SSZ = CHUNK // NS
PTS = N // NS
ZROWS = 8192
G = 128
TRASH = CHUNK

_mesh = plsc.VectorSubcoreMesh(
    core_axis_name="c", subcore_axis_name="s", num_cores=NC, num_subcores=NS
)


def _scalar(v16, i):
    return lax.squeeze(lax.slice(v16, (i,), (i + 1,)), (0,))


@functools.partial(
    pl.kernel,
    out_type=jax.ShapeDtypeStruct((E,), jnp.float32),
    mesh=_mesh,
    scratch_types=[
        pltpu.VMEM((PTS,), jnp.int32),
        pltpu.VMEM((PTS,), jnp.int32),
        pltpu.VMEM((PTS,), jnp.int32),
        pltpu.VMEM((PTS,), jnp.int32),
        pltpu.VMEM((PTS,), jnp.int32),
        pltpu.VMEM((CPS * L,), jnp.int32),
        pltpu.VMEM((CPS + L,), jnp.int32),
        pltpu.VMEM((G,), jnp.int32),
        pltpu.VMEM((G,), jnp.int32),
        pltpu.VMEM((KS, G), jnp.float32),
        pltpu.VMEM((ZROWS,), jnp.float32),
        pltpu.VMEM_SHARED((CHUNK + NS * 1024,), jnp.float32),
        pltpu.SemaphoreType.DMA,
        pltpu.SemaphoreType.DMA,
        pltpu.SemaphoreType.DMA,
        pltpu.SemaphoreType.DMA,
    ],
    compiler_params=pltpu.CompilerParams(needs_layout_passes=False),
)
def _scatter_kernel(xcol, ycol, vals1, out1, xs_v, ys_v, eb_v, vb_v, bli,
                    roff, binst, eidx, vidx, crows, zbuf, acc,
                    gsem, ssem, zsem, wsem):
    c = lax.axis_index("c")
    s = lax.axis_index("s")
    iota = lax.iota(jnp.int32, L)
    zero_i = jnp.zeros((L,), jnp.int32)
    zero_f = jnp.zeros((L,), jnp.float32)

    pltpu.sync_copy(xcol.at[pl.ds(s * PTS, PTS)], xs_v)
    pltpu.sync_copy(ycol.at[pl.ds(s * PTS, PTS)], ys_v)

    def zh(i, _):
        plsc.store_scatter(roff, [iota + i * L], zero_i)
        return 0

    lax.fori_loop(0, CPS, zh, 0)

    cb0 = c * CPS * CHUNK

    def mk_base(i, _):
        lanes = iota + i * L
        xs = plsc.load_gather(xs_v, [lanes])
        ys = plsc.load_gather(ys_v, [lanes])
        eb = xs * (W * KS) + ((ys >> 7) << 10) + (ys & 127)
        plsc.store_scatter(eb_v, [lanes], eb)
        p = lanes + s * PTS
        plsc.store_scatter(vb_v, [lanes], ((p >> 7) << 10) + (p & 127))
        cid = (eb - cb0) >> 20
        m = (cid >= 0) & (cid < CPS)
        hidx = (cid << 4) + iota
        h = plsc.load_gather(roff, [hidx], mask=m)
        plsc.store_scatter(roff, [hidx], h + 1, mask=m)
        return 0

    lax.fori_loop(0, PTS // L, mk_base, 0)

    base = jnp.int32(0)
    for b in range(CPS):
        plsc.store_scatter(binst, [zero_i + b], zero_i + base,
                           mask=(iota == 0))
        row = plsc.load_gather(roff, [zero_i + (b << 4) + iota])
        cs = plsc.cumsum(row)
        plsc.store_scatter(roff, [zero_i + (b << 4) + iota],
                           base + cs - row)
        base = base + _scalar(cs, L - 1)
    plsc.store_scatter(binst, [zero_i + CPS], zero_i + base,
                       mask=(iota == 0))

    def binify(i, _):
        lanes = iota + i * L
        eb = plsc.load_gather(eb_v, [lanes])
        cid = (eb - cb0) >> 20
        m = (cid >= 0) & (cid < CPS)
        hidx = (cid << 4) + iota
        pos = plsc.load_gather(roff, [hidx], mask=m)
        plsc.store_scatter(roff, [hidx], pos + 1, mask=m)
        plsc.store_scatter(bli, [pos], lanes, mask=m)
        return 0

    lax.fori_loop(0, PTS // L, binify, 0)

    def mk_zero(i, _):
        plsc.store_scatter(zbuf, [iota + i * L], zero_f)
        return 0

    lax.fori_loop(0, ZROWS // L, mk_zero, 0)

    def z0(j, _):
        pltpu.sync_copy(zbuf, acc.at[pl.ds(s * SSZ + j * ZROWS, ZROWS)])
        return 0

    lax.fori_loop(0, SSZ // ZROWS, z0, 0)

    pltpu.async_copy(acc.at[pl.ds(s * SSZ, SSZ)],
                     out1.at[pl.ds(cb0 + s * SSZ, SSZ)], wsem)

    ACCL = CHUNK + NS * 1024
    KPAD = (KS - 1) * G

    def expand(start, end, cb, g):
        for j in range(G // L):
            lanes = start + g * G + j * L + iota
            valid = lanes < end
            li = plsc.load_gather(bli, [lanes], mask=valid)
            e16 = plsc.load_gather(eb_v, [li], mask=valid) - cb
            v16 = plsc.load_gather(vb_v, [li], mask=valid)
            e16 = jnp.where(valid, e16, TRASH + s * 1024)
            v16 = jnp.where(valid, v16, s * (PTS * KS))
            plsc.store_scatter(eidx, [j * L + iota], e16)
            plsc.store_scatter(vidx, [j * L + iota], v16)

    def chunk_body(lc, _):
        cb = cb0 + lc * CHUNK
        st16 = plsc.load_gather(binst, [zero_i + lc])
        en16 = plsc.load_gather(binst, [zero_i + lc + 1])
        start, end = _scalar(st16, 0), _scalar(en16, 0)
        lcp = jnp.maximum(lc - 1, 0)
        pst16 = plsc.load_gather(binst, [zero_i + lcp])
        pen16 = plsc.load_gather(binst, [zero_i + lcp + 1])
        pstart, pend = _scalar(pst16, 0), _scalar(pen16, 0)
        pcnt = jnp.where(lc > 0, pend - pstart, 0)

        pltpu.make_async_copy(acc.at[pl.ds(s * SSZ, SSZ)],
                              out1.at[pl.ds(s * SSZ, SSZ)], wsem).wait()
        plsc.subcore_barrier()

        def zt(g, _):
            expand(pstart, pend, cb - CHUNK, g)
            for k in range(KS):
                pltpu.async_copy(
                    zbuf.at[pl.ds(0, G)],
                    acc.at[pl.ds(k * G, ACCL - KPAD)].at[eidx], zsem)
            for k in range(KS):
                pltpu.make_async_copy(
                    zbuf.at[pl.ds(0, G)],
                    acc.at[pl.ds(k * G, ACCL - KPAD)].at[eidx], zsem).wait()
            return 0

        lax.fori_loop(0, (pcnt + G - 1) >> 7, zt, 0)

        plsc.subcore_barrier()

        def accum(g, _):
            expand(start, end, cb, g)
            for k in range(KS):
                pltpu.async_copy(
                    vals1.at[pl.ds(k * G, VE - KPAD)].at[vidx],
                    crows.at[k], gsem)
            for k in range(KS):
                pltpu.make_async_copy(
                    vals1.at[pl.ds(k * G, VE - KPAD)].at[vidx],
                    crows.at[k], gsem).wait()
            for k in range(KS):
                pltpu.async_copy(
                    crows.at[k],
                    acc.at[pl.ds(k * G, ACCL - KPAD)].at[eidx], ssem,
                    add=True)
            for k in range(KS):
                pltpu.make_async_copy(
                    crows.at[k],
                    acc.at[pl.ds(k * G, ACCL - KPAD)].at[eidx], ssem).wait()
            return 0

        lax.fori_loop(0, (end - start + G - 1) >> 7, accum, 0)

        plsc.subcore_barrier()

        pltpu.async_copy(acc.at[pl.ds(s * SSZ, SSZ)],
                         out1.at[pl.ds(cb + s * SSZ, SSZ)], wsem)
        return 0

    lax.fori_loop(0, CPS, chunk_body, 0)

    pltpu.make_async_copy(acc.at[pl.ds(s * SSZ, SSZ)],
                          out1.at[pl.ds(s * SSZ, SSZ)], wsem).wait()


def kernel(coords, vals):
    xcol = coords[:, 0]
    ycol = coords[:, 1]
    vals1 = vals.reshape(N // 128, 128, KS).transpose(0, 2, 1).reshape(VE)
    out1 = _scatter_kernel(xcol, ycol, vals1)
    return out1.reshape(H, W // 128, KS, 128).transpose(0, 1, 3, 2).reshape(
        H, W, KS)
